# Initial kernel scaffold; baseline (speedup 1.0000x reference)
import jax
import jax.numpy as jnp
from jax.experimental import pallas as pl


def kernel(x, edge_index, edge_attr, game_state, hn, cn, W1, We1, as1, ad1, ae1, b1, W2, We2, as2, ad2, ae2, b2, W3, We3, as3, ad3, ae3, b3, P1, pb1, P2, pb2, gW, gb, R1, rb1, R2, rb2, Wih0, Whh0, bih0, bhh0, Wih1, Whh1, bih1, bhh1, cW, cb):
    raise NotImplementedError("write your pallas kernel here")


# speedup vs baseline: 60.9942x; 60.9942x over previous
import functools

import jax
import jax.numpy as jnp
from jax import lax
from jax.experimental import pallas as pl
from jax.experimental.pallas import tpu as pltpu
from jax.experimental.pallas import tpu_sc as plsc

N = 50000
E = 800000
NC = 2
NS = 16
B = 128
CPW = 196
EPW = CPW * B
EPAD = NC * NS * EPW
NPAD = 50048
RPT = NPAD // NS

_NEG = -1e30


def _make_sc_pass(combined):
    D = 16
    MW = 20
    Dh = 8
    mesh = plsc.VectorSubcoreMesh(core_axis_name="c", subcore_axis_name="s")

    nheads = 2 if combined else 1
    scratch = [
        pltpu.VMEM((CPW, B), jnp.int32),
        pltpu.VMEM((B,), jnp.int32),
        pltpu.VMEM((B, D), jnp.float32),
        pltpu.VMEM((B, MW), jnp.float32),
        pltpu.VMEM((2, B), jnp.float32),
    ]
    scratch += [pltpu.VMEM((B,), jnp.float32)] * (3 * nheads)
    scratch += [
        pltpu.VMEM_SHARED((NPAD, MW), jnp.float32),
        pltpu.SemaphoreType.DMA,
    ]

    def body(refs):
        if combined:
            (xg, als0, als1, ald0, ald1, ale0, ale1, srcs, dsts, zeros,
             out, dst_v, src_v, x_v, m_v, w_v,
             as0_v, as1_v, ad0_v, ad1_v, ale0_v, ale1_v, acc, sem) = refs
        else:
            (xg, als0, ald0, ale0, srcs, dsts, zeros,
             out, dst_v, src_v, x_v, m_v, w_v,
             as0_v, ad0_v, ale0_v, acc, sem) = refs
        c = lax.axis_index("c")
        s = lax.axis_index("s")
        iota16 = lax.iota(jnp.int32, 16)

        pltpu.sync_copy(zeros.at[pl.ds(s * RPT, RPT)],
                        acc.at[pl.ds(s * RPT, RPT)])
        pltpu.sync_copy(dsts.at[c, s], dst_v)
        plsc.subcore_barrier()

        ebase = (c * NS + s) * EPW

        def chunk(j, carry):
            pltpu.sync_copy(srcs.at[c, s, j], src_v)
            esl = pl.ds(ebase + j * B, B)
            pltpu.sync_copy(ale0.at[esl], ale0_v)
            if combined:
                pltpu.sync_copy(ale1.at[esl], ale1_v)
            dsl = dst_v.at[j]
            cps = [pltpu.async_copy(xg.at[src_v], x_v, sem),
                   pltpu.async_copy(als0.at[src_v], as0_v, sem),
                   pltpu.async_copy(ald0.at[dsl], ad0_v, sem)]
            if combined:
                cps.append(pltpu.async_copy(als1.at[src_v], as1_v, sem))
                cps.append(pltpu.async_copy(ald1.at[dsl], ad1_v, sem))
            for cp in cps:
                cp.wait()
            for g in range(B // 16):
                sl = pl.ds(g * 16, 16)
                e0 = as0_v[sl] + ad0_v[sl] + ale0_v[sl]
                e0 = jnp.where(e0 > 0, e0, 0.2 * e0)
                w_v[0, sl] = jnp.exp(e0)
                if combined:
                    e1 = as1_v[sl] + ad1_v[sl] + ale1_v[sl]
                    e1 = jnp.where(e1 > 0, e1, 0.2 * e1)
                    w_v[1, sl] = jnp.exp(e1)

            for g in range(B // 16):
                w0g = w_v[0, pl.ds(g * 16, 16)]
                w1g = w_v[1, pl.ds(g * 16, 16)] if combined else w0g
                for t in range(16):
                    i = g * 16 + t
                    if combined:
                        tail = jnp.where(iota16 == 12, w0g[t],
                                         jnp.where(iota16 == 13, w1g[t],
                                                   0.0))
                        wv = jnp.where(iota16 < Dh, w0g[t], w1g[t])
                    else:
                        tail = jnp.where(iota16 == 12, w0g[t], 0.0)
                        wv = w0g[t]
                    m_v[i, MW - 16:MW] = tail
                    m_v[i, 0:16] = wv * x_v[i, 0:16]

            pltpu.sync_copy(m_v, acc.at[dst_v.at[j]], add=True)
            return carry

        lax.fori_loop(0, CPW, chunk, 0)
        plsc.subcore_barrier()
        pltpu.sync_copy(acc.at[pl.ds(s * RPT, RPT)],
                        out.at[c, pl.ds(s * RPT, RPT)])

    @functools.partial(
        pl.kernel,
        mesh=mesh,
        compiler_params=pltpu.CompilerParams(use_tc_tiling_on_sc=False),
        out_type=jax.ShapeDtypeStruct((NC, NPAD, MW), jnp.float32),
        scratch_types=scratch,
    )
    def sc_pass(*refs):
        body(refs)

    return sc_pass


_sc_head = _make_sc_pass(False)
_sc_l3 = _make_sc_pass(True)


_BN = 1000
_NBLK = N // _BN
_BE = 4000
_EBLK = E // _BE


def _full(shape):
    return pl.BlockSpec(shape, lambda i: tuple(0 for _ in shape))


def _tc_nodes(x, W1, AsF, AdF):
    def body(x_ref, w_ref, as_ref, ad_ref, xw_ref, sa_ref):
        xw = jnp.dot(x_ref[...], w_ref[...], preferred_element_type=jnp.float32)
        xw_ref[...] = xw
        sal = jnp.dot(xw, as_ref[...], preferred_element_type=jnp.float32)
        sad = jnp.dot(xw, ad_ref[...], preferred_element_type=jnp.float32)
        sa_ref[...] = jnp.concatenate([sal, sad], axis=1)

    return pl.pallas_call(
        body,
        grid=(_NBLK,),
        in_specs=[
            pl.BlockSpec((_BN, 3), lambda i: (i, 0)),
            _full((3, 32)),
            _full((32, 2)),
            _full((32, 2)),
        ],
        out_specs=[
            pl.BlockSpec((_BN, 32), lambda i: (i, 0)),
            pl.BlockSpec((_BN, 4), lambda i: (i, 0)),
        ],
        out_shape=[
            jax.ShapeDtypeStruct((N, 32), jnp.float32),
            jax.ShapeDtypeStruct((N, 4), jnp.float32),
        ],
    )(x, W1, AsF, AdF)


def _tc_edges(ea, We1, ae1T, We2, ae2T, We3, ae3T):
    def body(ea_ref, w1_ref, a1_ref, w2_ref, a2_ref, w3_ref, a3_ref,
             o1_ref, o2_ref, o3_ref):
        eab = ea_ref[...]
        for w_ref, a_ref, o_ref, dh in ((w1_ref, a1_ref, o1_ref, 16),
                                        (w2_ref, a2_ref, o2_ref, 16),
                                        (w3_ref, a3_ref, o3_ref, 8)):
            we = w_ref[...]
            aT = a_ref[...]
            ce = jnp.concatenate(
                [jnp.dot(we[:, :dh], aT[:, 0:1],
                         preferred_element_type=jnp.float32),
                 jnp.dot(we[:, dh:], aT[:, 1:2],
                         preferred_element_type=jnp.float32)], axis=1)
            o_ref[...] = jnp.dot(eab, ce, preferred_element_type=jnp.float32)

    eblk = pl.BlockSpec((_BE, 2), lambda i: (i, 0))
    return pl.pallas_call(
        body,
        grid=(_EBLK,),
        in_specs=[
            eblk,
            _full((2, 32)), _full((16, 2)),
            _full((2, 32)), _full((16, 2)),
            _full((2, 16)), _full((8, 2)),
        ],
        out_specs=[eblk, eblk, eblk],
        out_shape=[jax.ShapeDtypeStruct((E, 2), jnp.float32)] * 3,
    )(ea, We1, ae1T, We2, ae2T, We3, ae3T)


def _tc_mid(accA, accB, bvec, Wn, AsF, AdF, Dn):
    def body(acca_ref, accb_ref, b_ref, w_ref, as_ref, ad_ref,
             xw_ref, sa_ref):
        pa = acca_ref[...]
        pb = accb_ref[...]
        sa_ = pa[0] + pa[1]
        sb_ = pb[0] + pb[1]
        h0 = sa_[:, :16] / (sa_[:, 16:17] + 1e-16)
        h1 = sb_[:, :16] / (sb_[:, 16:17] + 1e-16)
        o = jnp.concatenate([h0, h1], axis=1) + b_ref[...]
        h = jnp.where(o > 0, o, jnp.exp(o) - 1.0)
        xw = jnp.dot(h, w_ref[...], preferred_element_type=jnp.float32)
        xw_ref[...] = xw
        sal = jnp.dot(xw, as_ref[...], preferred_element_type=jnp.float32)
        sad = jnp.dot(xw, ad_ref[...], preferred_element_type=jnp.float32)
        sa_ref[...] = jnp.concatenate([sal, sad], axis=1)

    aspec = pl.BlockSpec((2, _BN, 20), lambda i: (0, i, 0))
    return pl.pallas_call(
        body,
        grid=(_NBLK,),
        in_specs=[
            aspec,
            aspec,
            _full((1, 32)),
            _full((32, Dn)),
            _full((Dn, 2)),
            _full((Dn, 2)),
        ],
        out_specs=[
            pl.BlockSpec((_BN, Dn), lambda i: (i, 0)),
            pl.BlockSpec((_BN, 4), lambda i: (i, 0)),
        ],
        out_shape=[
            jax.ShapeDtypeStruct((N, Dn), jnp.float32),
            jax.ShapeDtypeStruct((N, 4), jnp.float32),
        ],
    )(accA, accB, bvec, Wn, AsF, AdF)


def _tc_final(acc, b3, P1, pb1, P2, pb2, gW, gb, R1, rb1, R2, rb2,
              Wih0, Whh0, bih0, bhh0, Wih1, Whh1, bih1, bhh1,
              cW, cb, gs, hn, cn):
    def body(acc_ref, b_ref, p1_ref, pb1_ref, p2_ref, pb2_ref,
             gw_ref, gb_ref, r1_ref, rb1_ref, r2_ref, rb2_ref,
             wi0_ref, wh0_ref, bi0_ref, bh0_ref,
             wi1_ref, wh1_ref, bi1_ref, bh1_ref,
             cw_ref, cb_ref, gs_ref, hn_ref, cn_ref,
             out_ref, hn2_ref, cn2_ref, s_acc):
        i = pl.program_id(0)
        p = acc_ref[...]
        sm = p[0] + p[1]
        h0 = sm[:, :8] / (sm[:, 16:17] + 1e-16)
        h1 = sm[:, 8:16] / (sm[:, 17:18] + 1e-16)
        h = jnp.concatenate([h0, h1], axis=1) + b_ref[...]
        part = jnp.sum(h, axis=0, keepdims=True)

        @pl.when(i == 0)
        def _():
            s_acc[...] = part

        @pl.when(i > 0)
        def _():
            s_acc[...] = s_acc[...] + part

        @pl.when(i == _NBLK - 1)
        def _():
            def sig(v):
                return 1.0 / (1.0 + jnp.exp(-v))

            s = s_acc[...]
            t = jnp.dot(s, p1_ref[...],
                        preferred_element_type=jnp.float32) + pb1_ref[...]
            t = jnp.maximum(t, 0.0)
            pvec = jnp.dot(t, p2_ref[...],
                           preferred_element_type=jnp.float32) + pb2_ref[...]
            z = jnp.concatenate([pvec, gs_ref[...]], axis=1)
            z = jnp.dot(z, gw_ref[...],
                        preferred_element_type=jnp.float32) + gb_ref[...]
            r = jnp.maximum(jnp.dot(z, r1_ref[...],
                                    preferred_element_type=jnp.float32)
                            + rb1_ref[...], 0.0)
            z = z + jnp.dot(r, r2_ref[...],
                            preferred_element_type=jnp.float32) + rb2_ref[...]

            def cell(xt, h0v, c0v, wi, wh, bi, bh):
                g = (jnp.dot(xt, wi, preferred_element_type=jnp.float32)
                     + jnp.dot(h0v, wh, preferred_element_type=jnp.float32)
                     + bi + bh)
                ii = sig(g[:, 0:16])
                ff = sig(g[:, 16:32])
                gg = jnp.tanh(g[:, 32:48])
                oo = sig(g[:, 48:64])
                c2 = ff * c0v + ii * gg
                return oo * jnp.tanh(c2), c2

            hnb = hn_ref[...]
            cnb = cn_ref[...]
            h0n, c0n = cell(z, hnb[0:1], cnb[0:1],
                            wi0_ref[...], wh0_ref[...],
                            bi0_ref[...], bh0_ref[...])
            h1n, c1n = cell(h0n, hnb[1:2], cnb[1:2],
                            wi1_ref[...], wh1_ref[...],
                            bi1_ref[...], bh1_ref[...])
            logits = jnp.dot(h1n, cw_ref[...],
                             preferred_element_type=jnp.float32) + cb_ref[...]
            m = jnp.max(logits, axis=1, keepdims=True)
            ex = jnp.exp(logits - m)
            out_ref[...] = ex / jnp.sum(ex, axis=1, keepdims=True)
            hn2_ref[...] = jnp.concatenate([h0n, h1n], axis=0)
            cn2_ref[...] = jnp.concatenate([c0n, c1n], axis=0)

    return pl.pallas_call(
        body,
        grid=(_NBLK,),
        in_specs=[
            pl.BlockSpec((2, _BN, 20), lambda i: (0, i, 0)),
            _full((1, 16)),
            _full((16, 64)), _full((1, 64)),
            _full((64, 16)), _full((1, 16)),
            _full((20, 16)), _full((1, 16)),
            _full((16, 16)), _full((1, 16)),
            _full((16, 16)), _full((1, 16)),
            _full((16, 64)), _full((16, 64)), _full((1, 64)), _full((1, 64)),
            _full((16, 64)), _full((16, 64)), _full((1, 64)), _full((1, 64)),
            _full((16, 2)), _full((1, 2)),
            _full((1, 4)),
            _full((2, 16)), _full((2, 16)),
        ],
        out_specs=[_full((1, 2)), _full((2, 16)), _full((2, 16))],
        out_shape=[
            jax.ShapeDtypeStruct((1, 2), jnp.float32),
            jax.ShapeDtypeStruct((2, 16), jnp.float32),
            jax.ShapeDtypeStruct((2, 16), jnp.float32),
        ],
        scratch_shapes=[pltpu.VMEM((1, 16), jnp.float32)],
    )(acc, b3, P1, pb1, P2, pb2, gW, gb, R1, rb1, R2, rb2,
      Wih0, Whh0, bih0, bhh0, Wih1, Whh1, bih1, bhh1,
      cW, cb, gs, hn, cn)


def _blockdiag(a, dim):
    F = jnp.zeros((2 * dim, 2), jnp.float32)
    return F.at[:dim, 0].set(a[0]).at[dim:, 1].set(a[1])


def kernel(x, edge_index, edge_attr, game_state, hn, cn,
           W1, We1, as1, ad1, ae1, b1,
           W2, We2, as2, ad2, ae2, b2,
           W3, We3, as3, ad3, ae3, b3,
           P1, pb1, P2, pb2, gW, gb,
           R1, rb1, R2, rb2,
           Wih0, Whh0, bih0, bhh0, Wih1, Whh1, bih1, bhh1,
           cW, cb):
    ei = edge_index.astype(jnp.int32)
    pad = EPAD - E
    src = jnp.pad(ei[0], (0, pad)).reshape(NC, NS, CPW, B)
    dst = jnp.pad(ei[1], (0, pad)).reshape(NC, NS, CPW, B)

    ale1, ale2, ale3 = _tc_edges(edge_attr, We1, ae1.T, We2, ae2.T,
                                 We3, ae3.T)

    def esplit(ale):
        return (jnp.pad(ale[:, 0], (0, pad), constant_values=_NEG),
                jnp.pad(ale[:, 1], (0, pad), constant_values=_NEG))

    ale1_0, ale1_1 = esplit(ale1)
    ale2_0, ale2_1 = esplit(ale2)
    ale3_0, ale3_1 = esplit(ale3)

    z20 = jnp.zeros((NPAD, 20), jnp.float32)

    def layer12(xg, sa, ale_0, ale_1):
        a0 = _sc_head(xg[:, :16], sa[:, 0], sa[:, 2], ale_0,
                      src, dst, z20)[:, :N]
        a1 = _sc_head(xg[:, 16:], sa[:, 1], sa[:, 3], ale_1,
                      src, dst, z20)[:, :N]
        return a0, a1

    xg1, sa1 = _tc_nodes(x, W1, _blockdiag(as1, 16), _blockdiag(ad1, 16))
    a10, a11 = layer12(xg1, sa1, ale1_0, ale1_1)
    xg2, sa2 = _tc_mid(a10, a11, b1.reshape(1, 32), W2,
                       _blockdiag(as2, 16), _blockdiag(ad2, 16), 32)
    a20, a21 = layer12(xg2, sa2, ale2_0, ale2_1)
    xg3, sa3 = _tc_mid(a20, a21, b2.reshape(1, 32), W3,
                       _blockdiag(as3, 8), _blockdiag(ad3, 8), 16)
    acc3 = _sc_l3(xg3, sa3[:, 0], sa3[:, 1], sa3[:, 2], sa3[:, 3],
                  ale3_0, ale3_1, src, dst, z20)[:, :N]
    out, hn2, cn2 = _tc_final(
        acc3, b3.reshape(1, 16), P1, pb1.reshape(1, 64), P2,
        pb2.reshape(1, 16), gW, gb.reshape(1, 16), R1, rb1.reshape(1, 16),
        R2, rb2.reshape(1, 16), Wih0, Whh0, bih0.reshape(1, 64),
        bhh0.reshape(1, 64), Wih1, Whh1, bih1.reshape(1, 64),
        bhh1.reshape(1, 64), cW, cb.reshape(1, 2),
        game_state.reshape(1, 4), hn, cn)
    return (out.reshape(1, 2), hn2, cn2)


# speedup vs baseline: 70.7837x; 1.1605x over previous
import functools

import jax
import jax.numpy as jnp
from jax import lax
from jax.experimental import pallas as pl
from jax.experimental.pallas import tpu as pltpu
from jax.experimental.pallas import tpu_sc as plsc

N = 50000
E = 800000
NC = 2
NS = 16
B = 128
CPW = 196
EPW = CPW * B
EPAD = NC * NS * EPW
NPAD = 50048
RPT = NPAD // NS

_NEG = -1e30


def _make_sc_pass(combined):
    D = 16
    MW = 20
    Dh = 8
    mesh = plsc.VectorSubcoreMesh(core_axis_name="c", subcore_axis_name="s")

    nheads = 2 if combined else 1
    scratch = [
        pltpu.VMEM((CPW, B), jnp.int32),
    ]
    for _ in range(2):
        scratch += [
            pltpu.VMEM((B,), jnp.int32),
            pltpu.VMEM((B, D), jnp.float32),
            pltpu.VMEM((B, MW), jnp.float32),
            pltpu.VMEM((2, B), jnp.float32),
        ]
        scratch += [pltpu.VMEM((B,), jnp.float32)] * (3 * nheads)
        scratch += [pltpu.SemaphoreType.DMA]
    scratch += [
        pltpu.VMEM_SHARED((NPAD, MW), jnp.float32),
    ]

    def body(refs):
        if combined:
            (xg, als0, als1, ald0, ald1, ale0, ale1, srcs, dsts, zeros,
             out, dst_v,
             s_a, x_a, m_a, w_a, as0_a, as1_a, ad0_a, ad1_a, ae0_a, ae1_a,
             sem_a,
             s_b, x_b, m_b, w_b, as0_b, as1_b, ad0_b, ad1_b, ae0_b, ae1_b,
             sem_b,
             acc) = refs
            bufs = ((s_a, x_a, m_a, w_a, (as0_a, as1_a), (ad0_a, ad1_a),
                     (ae0_a, ae1_a), sem_a),
                    (s_b, x_b, m_b, w_b, (as0_b, as1_b), (ad0_b, ad1_b),
                     (ae0_b, ae1_b), sem_b))
            als = (als0, als1)
            ald = (ald0, ald1)
            alehbm = (ale0, ale1)
        else:
            (xg, als0, ald0, ale0, srcs, dsts, zeros,
             out, dst_v,
             s_a, x_a, m_a, w_a, as0_a, ad0_a, ae0_a, sem_a,
             s_b, x_b, m_b, w_b, as0_b, ad0_b, ae0_b, sem_b,
             acc) = refs
            bufs = ((s_a, x_a, m_a, w_a, (as0_a,), (ad0_a,), (ae0_a,),
                     sem_a),
                    (s_b, x_b, m_b, w_b, (as0_b,), (ad0_b,), (ae0_b,),
                     sem_b))
            als = (als0,)
            ald = (ald0,)
            alehbm = (ale0,)
        c = lax.axis_index("c")
        s = lax.axis_index("s")
        iota16 = lax.iota(jnp.int32, 16)

        pltpu.sync_copy(zeros.at[pl.ds(s * RPT, RPT)],
                        acc.at[pl.ds(s * RPT, RPT)])
        pltpu.sync_copy(dsts.at[c, s], dst_v)
        ebase = (c * NS + s) * EPW
        plsc.subcore_barrier()

        def issue(j, bsel):
            src_v, x_v, _, _, as_v, ad_v, ae_v, sem = bufs[bsel]
            pltpu.sync_copy(srcs.at[c, s, j], src_v)
            esl = pl.ds(ebase + j * B, B)
            for h in range(nheads):
                pltpu.sync_copy(alehbm[h].at[esl], ae_v[h])
            drow = dst_v.at[j]
            cps = [pltpu.async_copy(xg.at[src_v], x_v, sem)]
            for h in range(nheads):
                cps.append(pltpu.async_copy(als[h].at[src_v], as_v[h], sem))
                cps.append(pltpu.async_copy(ald[h].at[drow], ad_v[h], sem))
            return cps

        def consume(j, bsel, cps):
            _, x_v, m_v, w_v, as_v, ad_v, ae_v, sem = bufs[bsel]
            for cp in cps:
                cp.wait()
            for g in range(B // 16):
                sl = pl.ds(g * 16, 16)
                for h in range(nheads):
                    e = as_v[h][sl] + ad_v[h][sl] + ae_v[h][sl]
                    e = jnp.where(e > 0, e, 0.2 * e)
                    w_v[h, sl] = jnp.exp(e)

            for g in range(B // 16):
                w0g = w_v[0, pl.ds(g * 16, 16)]
                w1g = w_v[1, pl.ds(g * 16, 16)] if combined else w0g
                for t in range(16):
                    i = g * 16 + t
                    if combined:
                        tail = jnp.where(iota16 == 12, w0g[t],
                                         jnp.where(iota16 == 13, w1g[t],
                                                   0.0))
                        wv = jnp.where(iota16 < Dh, w0g[t], w1g[t])
                    else:
                        tail = jnp.where(iota16 == 12, w0g[t], 0.0)
                        wv = w0g[t]
                    m_v[i, MW - 16:MW] = tail
                    m_v[i, 0:16] = wv * x_v[i, 0:16]

            pltpu.sync_copy(m_v, acc.at[dst_v.at[j]], add=True)

        def pair(k, carry):
            j0 = 2 * k
            cps0 = issue(j0, 0)
            cps1 = issue(j0 + 1, 1)
            consume(j0, 0, cps0)
            consume(j0 + 1, 1, cps1)
            return carry

        lax.fori_loop(0, CPW // 2, pair, 0)
        plsc.subcore_barrier()
        pltpu.sync_copy(acc.at[pl.ds(s * RPT, RPT)],
                        out.at[c, pl.ds(s * RPT, RPT)])

    @functools.partial(
        pl.kernel,
        mesh=mesh,
        compiler_params=pltpu.CompilerParams(use_tc_tiling_on_sc=False),
        out_type=jax.ShapeDtypeStruct((NC, NPAD, MW), jnp.float32),
        scratch_types=scratch,
    )
    def sc_pass(*refs):
        body(refs)

    return sc_pass


_sc_head = _make_sc_pass(False)
_sc_l3 = _make_sc_pass(True)


_BN = 1000
_NBLK = N // _BN
_BE = 4000
_EBLK = E // _BE


def _full(shape):
    return pl.BlockSpec(shape, lambda i: tuple(0 for _ in shape))


def _tc_nodes(x, W1, AsF, AdF):
    def body(x_ref, w_ref, as_ref, ad_ref, xw_ref, sa_ref):
        xw = jnp.dot(x_ref[...], w_ref[...], preferred_element_type=jnp.float32)
        xw_ref[...] = xw
        sal = jnp.dot(xw, as_ref[...], preferred_element_type=jnp.float32)
        sad = jnp.dot(xw, ad_ref[...], preferred_element_type=jnp.float32)
        sa_ref[...] = jnp.concatenate([sal, sad], axis=1)

    return pl.pallas_call(
        body,
        grid=(_NBLK,),
        in_specs=[
            pl.BlockSpec((_BN, 3), lambda i: (i, 0)),
            _full((3, 32)),
            _full((32, 2)),
            _full((32, 2)),
        ],
        out_specs=[
            pl.BlockSpec((_BN, 32), lambda i: (i, 0)),
            pl.BlockSpec((_BN, 4), lambda i: (i, 0)),
        ],
        out_shape=[
            jax.ShapeDtypeStruct((N, 32), jnp.float32),
            jax.ShapeDtypeStruct((N, 4), jnp.float32),
        ],
    )(x, W1, AsF, AdF)


def _tc_edges(ea, We1, ae1T, We2, ae2T, We3, ae3T):
    def body(ea_ref, w1_ref, a1_ref, w2_ref, a2_ref, w3_ref, a3_ref,
             o1_ref, o2_ref, o3_ref):
        eab = ea_ref[...]
        for w_ref, a_ref, o_ref, dh in ((w1_ref, a1_ref, o1_ref, 16),
                                        (w2_ref, a2_ref, o2_ref, 16),
                                        (w3_ref, a3_ref, o3_ref, 8)):
            we = w_ref[...]
            aT = a_ref[...]
            ce = jnp.concatenate(
                [jnp.dot(we[:, :dh], aT[:, 0:1],
                         preferred_element_type=jnp.float32),
                 jnp.dot(we[:, dh:], aT[:, 1:2],
                         preferred_element_type=jnp.float32)], axis=1)
            o_ref[...] = jnp.dot(eab, ce, preferred_element_type=jnp.float32)

    eblk = pl.BlockSpec((_BE, 2), lambda i: (i, 0))
    return pl.pallas_call(
        body,
        grid=(_EBLK,),
        in_specs=[
            eblk,
            _full((2, 32)), _full((16, 2)),
            _full((2, 32)), _full((16, 2)),
            _full((2, 16)), _full((8, 2)),
        ],
        out_specs=[eblk, eblk, eblk],
        out_shape=[jax.ShapeDtypeStruct((E, 2), jnp.float32)] * 3,
    )(ea, We1, ae1T, We2, ae2T, We3, ae3T)


def _tc_mid(accA, accB, bvec, Wn, AsF, AdF, Dn):
    def body(acca_ref, accb_ref, b_ref, w_ref, as_ref, ad_ref,
             xw_ref, sa_ref):
        pa = acca_ref[...]
        pb = accb_ref[...]
        sa_ = pa[0] + pa[1]
        sb_ = pb[0] + pb[1]
        h0 = sa_[:, :16] / (sa_[:, 16:17] + 1e-16)
        h1 = sb_[:, :16] / (sb_[:, 16:17] + 1e-16)
        o = jnp.concatenate([h0, h1], axis=1) + b_ref[...]
        h = jnp.where(o > 0, o, jnp.exp(o) - 1.0)
        xw = jnp.dot(h, w_ref[...], preferred_element_type=jnp.float32)
        xw_ref[...] = xw
        sal = jnp.dot(xw, as_ref[...], preferred_element_type=jnp.float32)
        sad = jnp.dot(xw, ad_ref[...], preferred_element_type=jnp.float32)
        sa_ref[...] = jnp.concatenate([sal, sad], axis=1)

    aspec = pl.BlockSpec((2, _BN, 20), lambda i: (0, i, 0))
    return pl.pallas_call(
        body,
        grid=(_NBLK,),
        in_specs=[
            aspec,
            aspec,
            _full((1, 32)),
            _full((32, Dn)),
            _full((Dn, 2)),
            _full((Dn, 2)),
        ],
        out_specs=[
            pl.BlockSpec((_BN, Dn), lambda i: (i, 0)),
            pl.BlockSpec((_BN, 4), lambda i: (i, 0)),
        ],
        out_shape=[
            jax.ShapeDtypeStruct((N, Dn), jnp.float32),
            jax.ShapeDtypeStruct((N, 4), jnp.float32),
        ],
    )(accA, accB, bvec, Wn, AsF, AdF)


def _tc_final(acc, b3, P1, pb1, P2, pb2, gW, gb, R1, rb1, R2, rb2,
              Wih0, Whh0, bih0, bhh0, Wih1, Whh1, bih1, bhh1,
              cW, cb, gs, hn, cn):
    def body(acc_ref, b_ref, p1_ref, pb1_ref, p2_ref, pb2_ref,
             gw_ref, gb_ref, r1_ref, rb1_ref, r2_ref, rb2_ref,
             wi0_ref, wh0_ref, bi0_ref, bh0_ref,
             wi1_ref, wh1_ref, bi1_ref, bh1_ref,
             cw_ref, cb_ref, gs_ref, hn_ref, cn_ref,
             out_ref, hn2_ref, cn2_ref, s_acc):
        i = pl.program_id(0)
        p = acc_ref[...]
        sm = p[0] + p[1]
        h0 = sm[:, :8] / (sm[:, 16:17] + 1e-16)
        h1 = sm[:, 8:16] / (sm[:, 17:18] + 1e-16)
        h = jnp.concatenate([h0, h1], axis=1) + b_ref[...]
        part = jnp.sum(h, axis=0, keepdims=True)

        @pl.when(i == 0)
        def _():
            s_acc[...] = part

        @pl.when(i > 0)
        def _():
            s_acc[...] = s_acc[...] + part

        @pl.when(i == _NBLK - 1)
        def _():
            def sig(v):
                return 1.0 / (1.0 + jnp.exp(-v))

            s = s_acc[...]
            t = jnp.dot(s, p1_ref[...],
                        preferred_element_type=jnp.float32) + pb1_ref[...]
            t = jnp.maximum(t, 0.0)
            pvec = jnp.dot(t, p2_ref[...],
                           preferred_element_type=jnp.float32) + pb2_ref[...]
            z = jnp.concatenate([pvec, gs_ref[...]], axis=1)
            z = jnp.dot(z, gw_ref[...],
                        preferred_element_type=jnp.float32) + gb_ref[...]
            r = jnp.maximum(jnp.dot(z, r1_ref[...],
                                    preferred_element_type=jnp.float32)
                            + rb1_ref[...], 0.0)
            z = z + jnp.dot(r, r2_ref[...],
                            preferred_element_type=jnp.float32) + rb2_ref[...]

            def cell(xt, h0v, c0v, wi, wh, bi, bh):
                g = (jnp.dot(xt, wi, preferred_element_type=jnp.float32)
                     + jnp.dot(h0v, wh, preferred_element_type=jnp.float32)
                     + bi + bh)
                ii = sig(g[:, 0:16])
                ff = sig(g[:, 16:32])
                gg = jnp.tanh(g[:, 32:48])
                oo = sig(g[:, 48:64])
                c2 = ff * c0v + ii * gg
                return oo * jnp.tanh(c2), c2

            hnb = hn_ref[...]
            cnb = cn_ref[...]
            h0n, c0n = cell(z, hnb[0:1], cnb[0:1],
                            wi0_ref[...], wh0_ref[...],
                            bi0_ref[...], bh0_ref[...])
            h1n, c1n = cell(h0n, hnb[1:2], cnb[1:2],
                            wi1_ref[...], wh1_ref[...],
                            bi1_ref[...], bh1_ref[...])
            logits = jnp.dot(h1n, cw_ref[...],
                             preferred_element_type=jnp.float32) + cb_ref[...]
            m = jnp.max(logits, axis=1, keepdims=True)
            ex = jnp.exp(logits - m)
            out_ref[...] = ex / jnp.sum(ex, axis=1, keepdims=True)
            hn2_ref[...] = jnp.concatenate([h0n, h1n], axis=0)
            cn2_ref[...] = jnp.concatenate([c0n, c1n], axis=0)

    return pl.pallas_call(
        body,
        grid=(_NBLK,),
        in_specs=[
            pl.BlockSpec((2, _BN, 20), lambda i: (0, i, 0)),
            _full((1, 16)),
            _full((16, 64)), _full((1, 64)),
            _full((64, 16)), _full((1, 16)),
            _full((20, 16)), _full((1, 16)),
            _full((16, 16)), _full((1, 16)),
            _full((16, 16)), _full((1, 16)),
            _full((16, 64)), _full((16, 64)), _full((1, 64)), _full((1, 64)),
            _full((16, 64)), _full((16, 64)), _full((1, 64)), _full((1, 64)),
            _full((16, 2)), _full((1, 2)),
            _full((1, 4)),
            _full((2, 16)), _full((2, 16)),
        ],
        out_specs=[_full((1, 2)), _full((2, 16)), _full((2, 16))],
        out_shape=[
            jax.ShapeDtypeStruct((1, 2), jnp.float32),
            jax.ShapeDtypeStruct((2, 16), jnp.float32),
            jax.ShapeDtypeStruct((2, 16), jnp.float32),
        ],
        scratch_shapes=[pltpu.VMEM((1, 16), jnp.float32)],
    )(acc, b3, P1, pb1, P2, pb2, gW, gb, R1, rb1, R2, rb2,
      Wih0, Whh0, bih0, bhh0, Wih1, Whh1, bih1, bhh1,
      cW, cb, gs, hn, cn)


def _blockdiag(a, dim):
    F = jnp.zeros((2 * dim, 2), jnp.float32)
    return F.at[:dim, 0].set(a[0]).at[dim:, 1].set(a[1])


def kernel(x, edge_index, edge_attr, game_state, hn, cn,
           W1, We1, as1, ad1, ae1, b1,
           W2, We2, as2, ad2, ae2, b2,
           W3, We3, as3, ad3, ae3, b3,
           P1, pb1, P2, pb2, gW, gb,
           R1, rb1, R2, rb2,
           Wih0, Whh0, bih0, bhh0, Wih1, Whh1, bih1, bhh1,
           cW, cb):
    ei = edge_index.astype(jnp.int32)
    pad = EPAD - E
    src = jnp.pad(ei[0], (0, pad)).reshape(NC, NS, CPW, B)
    dst = jnp.pad(ei[1], (0, pad)).reshape(NC, NS, CPW, B)

    ale1, ale2, ale3 = _tc_edges(edge_attr, We1, ae1.T, We2, ae2.T,
                                 We3, ae3.T)

    def esplit(ale):
        return (jnp.pad(ale[:, 0], (0, pad), constant_values=_NEG),
                jnp.pad(ale[:, 1], (0, pad), constant_values=_NEG))

    ale1_0, ale1_1 = esplit(ale1)
    ale2_0, ale2_1 = esplit(ale2)
    ale3_0, ale3_1 = esplit(ale3)

    z20 = jnp.zeros((NPAD, 20), jnp.float32)

    def layer12(xg, sa, ale_0, ale_1):
        a0 = _sc_head(xg[:, :16], sa[:, 0], sa[:, 2], ale_0,
                      src, dst, z20)[:, :N]
        a1 = _sc_head(xg[:, 16:], sa[:, 1], sa[:, 3], ale_1,
                      src, dst, z20)[:, :N]
        return a0, a1

    xg1, sa1 = _tc_nodes(x, W1, _blockdiag(as1, 16), _blockdiag(ad1, 16))
    a10, a11 = layer12(xg1, sa1, ale1_0, ale1_1)
    xg2, sa2 = _tc_mid(a10, a11, b1.reshape(1, 32), W2,
                       _blockdiag(as2, 16), _blockdiag(ad2, 16), 32)
    a20, a21 = layer12(xg2, sa2, ale2_0, ale2_1)
    xg3, sa3 = _tc_mid(a20, a21, b2.reshape(1, 32), W3,
                       _blockdiag(as3, 8), _blockdiag(ad3, 8), 16)
    acc3 = _sc_l3(xg3, sa3[:, 0], sa3[:, 1], sa3[:, 2], sa3[:, 3],
                  ale3_0, ale3_1, src, dst, z20)[:, :N]
    out, hn2, cn2 = _tc_final(
        acc3, b3.reshape(1, 16), P1, pb1.reshape(1, 64), P2,
        pb2.reshape(1, 16), gW, gb.reshape(1, 16), R1, rb1.reshape(1, 16),
        R2, rb2.reshape(1, 16), Wih0, Whh0, bih0.reshape(1, 64),
        bhh0.reshape(1, 64), Wih1, Whh1, bih1.reshape(1, 64),
        bhh1.reshape(1, 64), cW, cb.reshape(1, 2),
        game_state.reshape(1, 4), hn, cn)
    return (out.reshape(1, 2), hn2, cn2)


# speedup vs baseline: 77.1072x; 1.0893x over previous
import functools

import jax
import jax.numpy as jnp
from jax import lax
from jax.experimental import pallas as pl
from jax.experimental.pallas import tpu as pltpu
from jax.experimental.pallas import tpu_sc as plsc

N = 50000
E = 800000
NC = 2
NS = 16
B = 128
CPW = 196
EPW = CPW * B
EPAD = NC * NS * EPW
NPAD = 50048
RPT = NPAD // NS

_NEG = -1e30


def _make_sc_pass(combined):
    D = 16
    MW = 20
    Dh = 8
    mesh = plsc.VectorSubcoreMesh(core_axis_name="c", subcore_axis_name="s")

    nheads = 2 if combined else 1
    scratch = [
        pltpu.VMEM((CPW, B), jnp.int32),
    ]
    for _ in range(4):
        scratch += [
            pltpu.VMEM((B,), jnp.int32),
            pltpu.VMEM((B, D), jnp.float32),
            pltpu.VMEM((B, MW), jnp.float32),
            pltpu.VMEM((2, B), jnp.float32),
        ]
        scratch += [pltpu.VMEM((B,), jnp.float32)] * (3 * nheads)
        scratch += [pltpu.SemaphoreType.DMA]
    scratch += [
        pltpu.VMEM_SHARED((NPAD, MW), jnp.float32),
    ]

    def body(refs):
        if combined:
            (xg, als0, als1, ald0, ald1, ale0, ale1, srcs, dsts, zeros,
             out, dst_v, *rest) = refs
            acc = rest[-1]
            per = 11
            bufs = tuple(
                (r[0], r[1], r[2], r[3], (r[4], r[5]), (r[6], r[7]),
                 (r[8], r[9]), r[10])
                for r in (rest[b * per:(b + 1) * per] for b in range(4)))
            als = (als0, als1)
            ald = (ald0, ald1)
            alehbm = (ale0, ale1)
        else:
            (xg, als0, ald0, ale0, srcs, dsts, zeros,
             out, dst_v, *rest) = refs
            acc = rest[-1]
            per = 8
            bufs = tuple(
                (r[0], r[1], r[2], r[3], (r[4],), (r[5],), (r[6],), r[7])
                for r in (rest[b * per:(b + 1) * per] for b in range(4)))
            als = (als0,)
            ald = (ald0,)
            alehbm = (ale0,)
        c = lax.axis_index("c")
        s = lax.axis_index("s")
        iota16 = lax.iota(jnp.int32, 16)

        pltpu.sync_copy(zeros.at[pl.ds(s * RPT, RPT)],
                        acc.at[pl.ds(s * RPT, RPT)])
        pltpu.sync_copy(dsts.at[c, s], dst_v)
        ebase = (c * NS + s) * EPW
        plsc.subcore_barrier()

        def issue(j, bsel):
            src_v, x_v, _, _, as_v, ad_v, ae_v, sem = bufs[bsel]
            pltpu.sync_copy(srcs.at[c, s, j], src_v)
            esl = pl.ds(ebase + j * B, B)
            for h in range(nheads):
                pltpu.sync_copy(alehbm[h].at[esl], ae_v[h])
            drow = dst_v.at[j]
            cps = [pltpu.async_copy(xg.at[src_v], x_v, sem)]
            for h in range(nheads):
                cps.append(pltpu.async_copy(als[h].at[src_v], as_v[h], sem))
                cps.append(pltpu.async_copy(ald[h].at[drow], ad_v[h], sem))
            return cps

        def consume(j, bsel, cps):
            _, x_v, m_v, w_v, as_v, ad_v, ae_v, sem = bufs[bsel]
            for cp in cps:
                cp.wait()
            for g in range(B // 16):
                sl = pl.ds(g * 16, 16)
                for h in range(nheads):
                    e = as_v[h][sl] + ad_v[h][sl] + ae_v[h][sl]
                    e = jnp.where(e > 0, e, 0.2 * e)
                    w_v[h, sl] = jnp.exp(e)

            for g in range(B // 16):
                w0g = w_v[0, pl.ds(g * 16, 16)]
                w1g = w_v[1, pl.ds(g * 16, 16)] if combined else w0g
                for t in range(16):
                    i = g * 16 + t
                    if combined:
                        tail = jnp.where(iota16 == 12, w0g[t],
                                         jnp.where(iota16 == 13, w1g[t],
                                                   0.0))
                        wv = jnp.where(iota16 < Dh, w0g[t], w1g[t])
                    else:
                        tail = jnp.where(iota16 == 12, w0g[t], 0.0)
                        wv = w0g[t]
                    m_v[i, MW - 16:MW] = tail
                    m_v[i, 0:16] = wv * x_v[i, 0:16]

            pltpu.sync_copy(m_v, acc.at[dst_v.at[j]], add=True)

        def quad(k, carry):
            j0 = 4 * k
            cps = [issue(j0 + b, b) for b in range(4)]
            for b in range(4):
                consume(j0 + b, b, cps[b])
            return carry

        lax.fori_loop(0, CPW // 4, quad, 0)
        plsc.subcore_barrier()
        pltpu.sync_copy(acc.at[pl.ds(s * RPT, RPT)],
                        out.at[c, pl.ds(s * RPT, RPT)])

    @functools.partial(
        pl.kernel,
        mesh=mesh,
        compiler_params=pltpu.CompilerParams(use_tc_tiling_on_sc=False),
        out_type=jax.ShapeDtypeStruct((NC, NPAD, MW), jnp.float32),
        scratch_types=scratch,
    )
    def sc_pass(*refs):
        body(refs)

    return sc_pass


_sc_head = _make_sc_pass(False)
_sc_l3 = _make_sc_pass(True)


_BN = 1000
_NBLK = N // _BN
_BE = 4000
_EBLK = E // _BE


def _full(shape):
    return pl.BlockSpec(shape, lambda i: tuple(0 for _ in shape))


def _tc_nodes(x, W1, AsF, AdF):
    def body(x_ref, w_ref, as_ref, ad_ref, xw_ref, sa_ref):
        xw = jnp.dot(x_ref[...], w_ref[...], preferred_element_type=jnp.float32)
        xw_ref[...] = xw
        sal = jnp.dot(xw, as_ref[...], preferred_element_type=jnp.float32)
        sad = jnp.dot(xw, ad_ref[...], preferred_element_type=jnp.float32)
        sa_ref[...] = jnp.concatenate([sal, sad], axis=1)

    return pl.pallas_call(
        body,
        grid=(_NBLK,),
        in_specs=[
            pl.BlockSpec((_BN, 3), lambda i: (i, 0)),
            _full((3, 32)),
            _full((32, 2)),
            _full((32, 2)),
        ],
        out_specs=[
            pl.BlockSpec((_BN, 32), lambda i: (i, 0)),
            pl.BlockSpec((_BN, 4), lambda i: (i, 0)),
        ],
        out_shape=[
            jax.ShapeDtypeStruct((N, 32), jnp.float32),
            jax.ShapeDtypeStruct((N, 4), jnp.float32),
        ],
    )(x, W1, AsF, AdF)


def _tc_edges(ea, We1, ae1T, We2, ae2T, We3, ae3T):
    def body(ea_ref, w1_ref, a1_ref, w2_ref, a2_ref, w3_ref, a3_ref,
             o1_ref, o2_ref, o3_ref):
        eab = ea_ref[...]
        for w_ref, a_ref, o_ref, dh in ((w1_ref, a1_ref, o1_ref, 16),
                                        (w2_ref, a2_ref, o2_ref, 16),
                                        (w3_ref, a3_ref, o3_ref, 8)):
            we = w_ref[...]
            aT = a_ref[...]
            ce = jnp.concatenate(
                [jnp.dot(we[:, :dh], aT[:, 0:1],
                         preferred_element_type=jnp.float32),
                 jnp.dot(we[:, dh:], aT[:, 1:2],
                         preferred_element_type=jnp.float32)], axis=1)
            o_ref[...] = jnp.dot(eab, ce, preferred_element_type=jnp.float32)

    eblk = pl.BlockSpec((_BE, 2), lambda i: (i, 0))
    return pl.pallas_call(
        body,
        grid=(_EBLK,),
        in_specs=[
            eblk,
            _full((2, 32)), _full((16, 2)),
            _full((2, 32)), _full((16, 2)),
            _full((2, 16)), _full((8, 2)),
        ],
        out_specs=[eblk, eblk, eblk],
        out_shape=[jax.ShapeDtypeStruct((E, 2), jnp.float32)] * 3,
    )(ea, We1, ae1T, We2, ae2T, We3, ae3T)


def _tc_mid(accA, accB, bvec, Wn, AsF, AdF, Dn):
    def body(acca_ref, accb_ref, b_ref, w_ref, as_ref, ad_ref,
             xw_ref, sa_ref):
        pa = acca_ref[...]
        pb = accb_ref[...]
        sa_ = pa[0] + pa[1]
        sb_ = pb[0] + pb[1]
        h0 = sa_[:, :16] / (sa_[:, 16:17] + 1e-16)
        h1 = sb_[:, :16] / (sb_[:, 16:17] + 1e-16)
        o = jnp.concatenate([h0, h1], axis=1) + b_ref[...]
        h = jnp.where(o > 0, o, jnp.exp(o) - 1.0)
        xw = jnp.dot(h, w_ref[...], preferred_element_type=jnp.float32)
        xw_ref[...] = xw
        sal = jnp.dot(xw, as_ref[...], preferred_element_type=jnp.float32)
        sad = jnp.dot(xw, ad_ref[...], preferred_element_type=jnp.float32)
        sa_ref[...] = jnp.concatenate([sal, sad], axis=1)

    aspec = pl.BlockSpec((2, _BN, 20), lambda i: (0, i, 0))
    return pl.pallas_call(
        body,
        grid=(_NBLK,),
        in_specs=[
            aspec,
            aspec,
            _full((1, 32)),
            _full((32, Dn)),
            _full((Dn, 2)),
            _full((Dn, 2)),
        ],
        out_specs=[
            pl.BlockSpec((_BN, Dn), lambda i: (i, 0)),
            pl.BlockSpec((_BN, 4), lambda i: (i, 0)),
        ],
        out_shape=[
            jax.ShapeDtypeStruct((N, Dn), jnp.float32),
            jax.ShapeDtypeStruct((N, 4), jnp.float32),
        ],
    )(accA, accB, bvec, Wn, AsF, AdF)


def _tc_final(acc, b3, P1, pb1, P2, pb2, gW, gb, R1, rb1, R2, rb2,
              Wih0, Whh0, bih0, bhh0, Wih1, Whh1, bih1, bhh1,
              cW, cb, gs, hn, cn):
    def body(acc_ref, b_ref, p1_ref, pb1_ref, p2_ref, pb2_ref,
             gw_ref, gb_ref, r1_ref, rb1_ref, r2_ref, rb2_ref,
             wi0_ref, wh0_ref, bi0_ref, bh0_ref,
             wi1_ref, wh1_ref, bi1_ref, bh1_ref,
             cw_ref, cb_ref, gs_ref, hn_ref, cn_ref,
             out_ref, hn2_ref, cn2_ref, s_acc):
        i = pl.program_id(0)
        p = acc_ref[...]
        sm = p[0] + p[1]
        h0 = sm[:, :8] / (sm[:, 16:17] + 1e-16)
        h1 = sm[:, 8:16] / (sm[:, 17:18] + 1e-16)
        h = jnp.concatenate([h0, h1], axis=1) + b_ref[...]
        part = jnp.sum(h, axis=0, keepdims=True)

        @pl.when(i == 0)
        def _():
            s_acc[...] = part

        @pl.when(i > 0)
        def _():
            s_acc[...] = s_acc[...] + part

        @pl.when(i == _NBLK - 1)
        def _():
            def sig(v):
                return 1.0 / (1.0 + jnp.exp(-v))

            s = s_acc[...]
            t = jnp.dot(s, p1_ref[...],
                        preferred_element_type=jnp.float32) + pb1_ref[...]
            t = jnp.maximum(t, 0.0)
            pvec = jnp.dot(t, p2_ref[...],
                           preferred_element_type=jnp.float32) + pb2_ref[...]
            z = jnp.concatenate([pvec, gs_ref[...]], axis=1)
            z = jnp.dot(z, gw_ref[...],
                        preferred_element_type=jnp.float32) + gb_ref[...]
            r = jnp.maximum(jnp.dot(z, r1_ref[...],
                                    preferred_element_type=jnp.float32)
                            + rb1_ref[...], 0.0)
            z = z + jnp.dot(r, r2_ref[...],
                            preferred_element_type=jnp.float32) + rb2_ref[...]

            def cell(xt, h0v, c0v, wi, wh, bi, bh):
                g = (jnp.dot(xt, wi, preferred_element_type=jnp.float32)
                     + jnp.dot(h0v, wh, preferred_element_type=jnp.float32)
                     + bi + bh)
                ii = sig(g[:, 0:16])
                ff = sig(g[:, 16:32])
                gg = jnp.tanh(g[:, 32:48])
                oo = sig(g[:, 48:64])
                c2 = ff * c0v + ii * gg
                return oo * jnp.tanh(c2), c2

            hnb = hn_ref[...]
            cnb = cn_ref[...]
            h0n, c0n = cell(z, hnb[0:1], cnb[0:1],
                            wi0_ref[...], wh0_ref[...],
                            bi0_ref[...], bh0_ref[...])
            h1n, c1n = cell(h0n, hnb[1:2], cnb[1:2],
                            wi1_ref[...], wh1_ref[...],
                            bi1_ref[...], bh1_ref[...])
            logits = jnp.dot(h1n, cw_ref[...],
                             preferred_element_type=jnp.float32) + cb_ref[...]
            m = jnp.max(logits, axis=1, keepdims=True)
            ex = jnp.exp(logits - m)
            out_ref[...] = ex / jnp.sum(ex, axis=1, keepdims=True)
            hn2_ref[...] = jnp.concatenate([h0n, h1n], axis=0)
            cn2_ref[...] = jnp.concatenate([c0n, c1n], axis=0)

    return pl.pallas_call(
        body,
        grid=(_NBLK,),
        in_specs=[
            pl.BlockSpec((2, _BN, 20), lambda i: (0, i, 0)),
            _full((1, 16)),
            _full((16, 64)), _full((1, 64)),
            _full((64, 16)), _full((1, 16)),
            _full((20, 16)), _full((1, 16)),
            _full((16, 16)), _full((1, 16)),
            _full((16, 16)), _full((1, 16)),
            _full((16, 64)), _full((16, 64)), _full((1, 64)), _full((1, 64)),
            _full((16, 64)), _full((16, 64)), _full((1, 64)), _full((1, 64)),
            _full((16, 2)), _full((1, 2)),
            _full((1, 4)),
            _full((2, 16)), _full((2, 16)),
        ],
        out_specs=[_full((1, 2)), _full((2, 16)), _full((2, 16))],
        out_shape=[
            jax.ShapeDtypeStruct((1, 2), jnp.float32),
            jax.ShapeDtypeStruct((2, 16), jnp.float32),
            jax.ShapeDtypeStruct((2, 16), jnp.float32),
        ],
        scratch_shapes=[pltpu.VMEM((1, 16), jnp.float32)],
    )(acc, b3, P1, pb1, P2, pb2, gW, gb, R1, rb1, R2, rb2,
      Wih0, Whh0, bih0, bhh0, Wih1, Whh1, bih1, bhh1,
      cW, cb, gs, hn, cn)


def _blockdiag(a, dim):
    F = jnp.zeros((2 * dim, 2), jnp.float32)
    return F.at[:dim, 0].set(a[0]).at[dim:, 1].set(a[1])


def kernel(x, edge_index, edge_attr, game_state, hn, cn,
           W1, We1, as1, ad1, ae1, b1,
           W2, We2, as2, ad2, ae2, b2,
           W3, We3, as3, ad3, ae3, b3,
           P1, pb1, P2, pb2, gW, gb,
           R1, rb1, R2, rb2,
           Wih0, Whh0, bih0, bhh0, Wih1, Whh1, bih1, bhh1,
           cW, cb):
    ei = edge_index.astype(jnp.int32)
    pad = EPAD - E
    src = jnp.pad(ei[0], (0, pad)).reshape(NC, NS, CPW, B)
    dst = jnp.pad(ei[1], (0, pad)).reshape(NC, NS, CPW, B)

    ale1, ale2, ale3 = _tc_edges(edge_attr, We1, ae1.T, We2, ae2.T,
                                 We3, ae3.T)

    def esplit(ale):
        return (jnp.pad(ale[:, 0], (0, pad), constant_values=_NEG),
                jnp.pad(ale[:, 1], (0, pad), constant_values=_NEG))

    ale1_0, ale1_1 = esplit(ale1)
    ale2_0, ale2_1 = esplit(ale2)
    ale3_0, ale3_1 = esplit(ale3)

    z20 = jnp.zeros((NPAD, 20), jnp.float32)

    def layer12(xg, sa, ale_0, ale_1):
        a0 = _sc_head(xg[:, :16], sa[:, 0], sa[:, 2], ale_0,
                      src, dst, z20)[:, :N]
        a1 = _sc_head(xg[:, 16:], sa[:, 1], sa[:, 3], ale_1,
                      src, dst, z20)[:, :N]
        return a0, a1

    xg1, sa1 = _tc_nodes(x, W1, _blockdiag(as1, 16), _blockdiag(ad1, 16))
    a10, a11 = layer12(xg1, sa1, ale1_0, ale1_1)
    xg2, sa2 = _tc_mid(a10, a11, b1.reshape(1, 32), W2,
                       _blockdiag(as2, 16), _blockdiag(ad2, 16), 32)
    a20, a21 = layer12(xg2, sa2, ale2_0, ale2_1)
    xg3, sa3 = _tc_mid(a20, a21, b2.reshape(1, 32), W3,
                       _blockdiag(as3, 8), _blockdiag(ad3, 8), 16)
    acc3 = _sc_l3(xg3, sa3[:, 0], sa3[:, 1], sa3[:, 2], sa3[:, 3],
                  ale3_0, ale3_1, src, dst, z20)[:, :N]
    out, hn2, cn2 = _tc_final(
        acc3, b3.reshape(1, 16), P1, pb1.reshape(1, 64), P2,
        pb2.reshape(1, 16), gW, gb.reshape(1, 16), R1, rb1.reshape(1, 16),
        R2, rb2.reshape(1, 16), Wih0, Whh0, bih0.reshape(1, 64),
        bhh0.reshape(1, 64), Wih1, Whh1, bih1.reshape(1, 64),
        bhh1.reshape(1, 64), cW, cb.reshape(1, 2),
        game_state.reshape(1, 4), hn, cn)
    return (out.reshape(1, 2), hn2, cn2)
